# baseline (speedup 1.0000x reference)
import functools

import jax
import jax.numpy as jnp
from jax import lax
from jax.experimental import pallas as pl
from jax.experimental.pallas import tpu as pltpu
from jax.experimental.pallas import tpu_sc as plsc

D = 768
E = 8
T = 2048
H = 4 * D

TM = 256
FM = 512
NT = T // FM
H_T = 3072
NH = H // H_T
MAX_PAIRS = NT + E - 1
G = NH * MAX_PAIRS


def _router_body(x_ref, gw_ref, gb_ref, pos_ref, offs_ref):
    f32 = jnp.float32
    x = x_ref[...]
    gw = gw_ref[...]
    logits = lax.dot_general(gw, x, (((1,), (1,)), ((), ())),
                             preferred_element_type=f32) + gb_ref[...]
    m = jnp.max(logits, axis=0, keepdims=True)
    ex = jnp.exp(logits - m)
    scores = ex / jnp.sum(ex, axis=0, keepdims=True)
    smax = jnp.max(scores, axis=0, keepdims=True)
    eids = lax.broadcasted_iota(jnp.int32, (E, 1), 0)
    assign = jnp.min(jnp.where(scores == smax, eids, E), axis=0, keepdims=True)
    onehot = (assign == eids).astype(f32)

    counts = jnp.sum(onehot, axis=1, keepdims=True)
    r8 = lax.broadcasted_iota(jnp.int32, (E, E), 0)
    c8 = lax.broadcasted_iota(jnp.int32, (E, E), 1)
    lower = (r8 > c8).astype(f32)
    offs_e = lax.dot_general(lower, counts, (((1,), (0,)), ((), ())),
                             precision=lax.Precision.HIGHEST,
                             preferred_element_type=f32)
    offs_tok = jnp.sum(onehot * offs_e, axis=0, keepdims=True)

    rl = lax.broadcasted_iota(jnp.int32, (TM, TM), 0)
    cl = lax.broadcasted_iota(jnp.int32, (TM, TM), 1)
    triu = (rl <= cl).astype(f32)
    carry = jnp.zeros((E, 1), f32)
    for c in range(T // TM):
        blk = onehot[:, c * TM:(c + 1) * TM]
        csum = lax.dot_general(blk, triu, (((1,), (0,)), ((), ())),
                               preferred_element_type=f32) + carry
        rank_in = jnp.sum(csum * blk, axis=0, keepdims=True)
        pos_blk = offs_tok[:, c * TM:(c + 1) * TM] + rank_in - 1.0
        pos_ref[:, c * TM:(c + 1) * TM] = pos_blk.astype(jnp.int32)
        carry = carry + jnp.sum(blk, axis=1, keepdims=True)

    offs_ref[...] = offs_e.astype(jnp.int32)


def _run_router(x, gate_w, gate_b):
    pos, offs = pl.pallas_call(
        _router_body,
        out_shape=(
            jax.ShapeDtypeStruct((1, T), jnp.int32),
            jax.ShapeDtypeStruct((E, 1), jnp.int32),
        ),
    )(x, gate_w, gate_b.reshape(E, 1))
    offsets = jnp.concatenate([offs.reshape(E), jnp.full((1,), T, jnp.int32)])
    return pos.reshape(T), offsets


_NC, _NS = 2, 16
_NW = _NC * _NS
_CH = T // _NW


def _sc_gather_body(src_hbm, idx_hbm, out_hbm, idx_v, rows_v, sem):
    wid = lax.axis_index("s") * _NC + lax.axis_index("c")
    base = wid * _CH
    pltpu.sync_copy(idx_hbm.at[pl.ds(base, _CH)], idx_v)
    pltpu.async_copy(src_hbm.at[idx_v], rows_v, sem).wait()
    pltpu.sync_copy(rows_v, out_hbm.at[pl.ds(base, _CH)])


def _sc_scatter_body(src_hbm, idx_hbm, out_hbm, idx_v, rows_v, sem):
    wid = lax.axis_index("s") * _NC + lax.axis_index("c")
    base = wid * _CH
    pltpu.sync_copy(idx_hbm.at[pl.ds(base, _CH)], idx_v)
    pltpu.sync_copy(src_hbm.at[pl.ds(base, _CH)], rows_v)
    pltpu.async_copy(rows_v, out_hbm.at[idx_v], sem).wait()


def _sc_rows(body, src, idx):
    mesh = plsc.VectorSubcoreMesh(core_axis_name="c", subcore_axis_name="s")
    return pl.kernel(
        body,
        mesh=mesh,
        out_type=jax.ShapeDtypeStruct((T, D), jnp.float32),
        scratch_types=[
            pltpu.VMEM((_CH,), jnp.int32),
            pltpu.VMEM((_CH, D), jnp.float32),
            pltpu.SemaphoreType.DMA,
        ],
    )(src, idx)


def _unit_metadata(offsets):
    i32 = jnp.int32
    offs = offsets.astype(i32)
    counts = offs[1:] - offs[:-1]
    first_t = offs[:-1] // FM
    last_t = jnp.maximum(offs[1:] - 1, 0) // FM
    ntiles = jnp.where(counts > 0, last_t - first_t + 1, 0)
    base = jnp.concatenate([jnp.zeros((1,), i32), jnp.cumsum(ntiles)])
    unit_base = NH * base
    total = unit_base[E]
    g = jnp.arange(G, dtype=i32)
    e_g = jnp.minimum(jnp.sum(g[:, None] >= unit_base[None, 1:], axis=1,
                              dtype=i32), E - 1)
    r = g - unit_base[e_g]
    nt = jnp.maximum(ntiles[e_g], 1)
    h_g = r // nt
    t_g = first_t[e_g] + r % nt
    act = (g < total)
    li = jnp.maximum(total - 1, 0)
    e_g = jnp.where(act, e_g, e_g[li])
    h_g = jnp.where(act, h_g, h_g[li])
    t_g = jnp.where(act, t_g, t_g[li])
    ini = jnp.concatenate([jnp.ones((1,), jnp.bool_), t_g[1:] != t_g[:-1]])
    return t_g, e_g, h_g, act.astype(i32), ini.astype(i32)


def _ffn_body(t_ref, e_ref, h_ref, a_ref, i_ref, offs_ref,
              x_ref, w1_ref, b1_ref, w2_ref, b2_ref, out_ref):
    g = pl.program_id(0)

    @pl.when(a_ref[g] == 1)
    def _work():
        t = t_ref[g]
        e = e_ref[g]
        h = h_ref[g]
        row0 = t * FM
        glo = jnp.maximum(offs_ref[e], row0)
        ghi = jnp.minimum(offs_ref[e + 1], row0 + FM)
        rid = row0 + lax.broadcasted_iota(jnp.int32, (FM, 1), 0)
        mask = (rid >= glo) & (rid < ghi)

        xt = x_ref[...]
        hid = lax.dot_general(xt, w1_ref[0], (((1,), (1,)), ((), ())),
                              preferred_element_type=jnp.float32)
        hid = jnp.maximum(hid + b1_ref[0], 0.0)
        part = lax.dot_general(hid, w2_ref[0], (((1,), (1,)), ((), ())),
                               preferred_element_type=jnp.float32)
        part = part + jnp.where(h == 0, b2_ref[0], jnp.zeros_like(b2_ref[0]))
        contrib = jnp.where(mask, part, 0.0)

        @pl.when(i_ref[g] == 1)
        def _first():
            out_ref[...] = contrib

        @pl.when(i_ref[g] == 0)
        def _accum():
            out_ref[...] = out_ref[...] + contrib


def _run_ffn(xs, w1, b1, w2, b2, offsets, meta):
    t_g, e_g, h_g, act, ini = meta
    grid_spec = pltpu.PrefetchScalarGridSpec(
        num_scalar_prefetch=6,
        grid=(G,),
        in_specs=[
            pl.BlockSpec((FM, D), lambda g, t, e, h, a, i, o: (t[g], 0)),
            pl.BlockSpec((1, H_T, D), lambda g, t, e, h, a, i, o: (e[g], h[g], 0)),
            pl.BlockSpec((1, 1, H_T), lambda g, t, e, h, a, i, o: (e[g] * NH + h[g], 0, 0)),
            pl.BlockSpec((1, D, H_T), lambda g, t, e, h, a, i, o: (e[g], 0, h[g])),
            pl.BlockSpec((1, 1, D), lambda g, t, e, h, a, i, o: (e[g], 0, 0)),
        ],
        out_specs=pl.BlockSpec((FM, D), lambda g, t, e, h, a, i, o: (t[g], 0)),
    )
    return pl.pallas_call(
        _ffn_body,
        grid_spec=grid_spec,
        out_shape=jax.ShapeDtypeStruct((T, D), jnp.float32),
        compiler_params=pltpu.CompilerParams(
            dimension_semantics=("arbitrary",)),
    )(t_g, e_g, h_g, act, ini, offsets, xs, w1,
      b1.reshape(E * NH, 1, H_T), w2, b2.reshape(E, 1, D))


def kernel(x, gate_w, gate_b, w1, b1, w2, b2):
    pos, offsets = _run_router(x, gate_w, gate_b)
    meta = _unit_metadata(offsets)
    xs = _sc_rows(_sc_scatter_body, x, pos)
    ys = _run_ffn(xs, w1, b1, w2, b2, offsets, meta)
    return _sc_rows(_sc_gather_body, ys, pos)
